# baseline (speedup 1.0000x reference)
import functools

import jax
import jax.numpy as jnp
from jax import lax
from jax.experimental import pallas as pl
from jax.experimental.pallas import tpu as pltpu
from jax.experimental.pallas import tpu_sc as plsc

_VOCAB = 1_000_000
_D = 64
_NC, _NS, _L = 2, 16, 16
_NW = _NC * _NS
_BT = 16384 * 50
_BPW = _BT // _NW
_C = 256
_NCHUNK = _BPW // _C
_NBUF = 4
_NGRP = _NCHUNK // _NBUF

_mesh = plsc.VectorSubcoreMesh(core_axis_name="c", subcore_axis_name="s")


@functools.partial(
    pl.kernel,
    mesh=_mesh,
    out_type=jax.ShapeDtypeStruct((_BT, _D), jnp.float32),
    compiler_params=pltpu.CompilerParams(use_tc_tiling_on_sc=False),
    scratch_types=[
        pltpu.VMEM((_BPW,), jnp.int32),
        pltpu.VMEM((_C, _D), jnp.float32),
        pltpu.VMEM((_C, _D), jnp.float32),
        pltpu.VMEM((_C, _D), jnp.float32),
        pltpu.VMEM((_C, _D), jnp.float32),
        pltpu.SemaphoreType.DMA,
        pltpu.SemaphoreType.DMA,
        pltpu.SemaphoreType.DMA,
        pltpu.SemaphoreType.DMA,
        pltpu.SemaphoreType.DMA,
        pltpu.SemaphoreType.DMA,
        pltpu.SemaphoreType.DMA,
        pltpu.SemaphoreType.DMA,
    ],
)
def _emb_lookup(token_hbm, table_hbm, out_hbm, idx_v,
                b0, b1, b2, b3, g0, g1, g2, g3, s0, s1, s2, s3):
    bufs = (b0, b1, b2, b3)
    gsem = (g0, g1, g2, g3)
    ssem = (s0, s1, s2, s3)
    wid = lax.axis_index("s") * _NC + lax.axis_index("c")
    base = wid * _BPW
    _HEAD = (_NBUF - 1) * _C
    pltpu.async_copy(token_hbm.at[pl.ds(base, _HEAD)],
                     idx_v.at[pl.ds(0, _HEAD)], g3)
    pltpu.async_copy(token_hbm.at[pl.ds(base + _HEAD, _BPW - _HEAD)],
                     idx_v.at[pl.ds(_HEAD, _BPW - _HEAD)], s3)
    pltpu.make_async_copy(token_hbm.at[pl.ds(base, _HEAD)],
                          idx_v.at[pl.ds(0, _HEAD)], g3).wait()

    def clamp_span(lo, hi):
        def cb(j, _):
            s = j * _L
            v = idx_v[pl.ds(s, _L)]
            idx_v[pl.ds(s, _L)] = jnp.minimum(jnp.maximum(v, 0), _VOCAB - 1)
            return 0
        lax.fori_loop(lo // _L, hi // _L, cb, 0)

    def fire_gather(c, buf, sem):
        pltpu.async_copy(table_hbm.at[idx_v.at[pl.ds(c * _C, _C)]], buf, sem)

    def wait_gather(buf, sem):
        pltpu.make_async_copy(table_hbm.at[idx_v.at[pl.ds(0, _C)]], buf, sem).wait()

    def fire_store(c, buf, sem):
        pltpu.async_copy(buf, out_hbm.at[pl.ds(base + c * _C, _C)], sem)

    def wait_store(buf, sem):
        pltpu.make_async_copy(buf, out_hbm.at[pl.ds(base, _C)], sem).wait()

    clamp_span(0, _HEAD)
    for b in range(_NBUF - 1):
        fire_gather(b, bufs[b], gsem[b])
    pltpu.make_async_copy(token_hbm.at[pl.ds(base + _HEAD, _BPW - _HEAD)],
                          idx_v.at[pl.ds(_HEAD, _BPW - _HEAD)], s3).wait()
    clamp_span(_HEAD, _BPW)

    def body(g, _):
        i0 = g * _NBUF
        for b in range(_NBUF):
            i = i0 + b
            b2 = (b + _NBUF - 1) % _NBUF
            ahead = i + _NBUF - 1

            @pl.when(jnp.logical_and(ahead < _NCHUNK, i >= 1))
            def _drain_prev_store():
                wait_store(bufs[b2], ssem[b2])

            @pl.when(ahead < _NCHUNK)
            def _launch_ahead():
                fire_gather(ahead, bufs[b2], gsem[b2])

            wait_gather(bufs[b], gsem[b])
            fire_store(i, bufs[b], ssem[b])
        return 0

    lax.fori_loop(0, _NGRP, body, 0)
    for b in range(_NBUF):
        wait_store(bufs[b], ssem[b])


def kernel(token, table):
    out = _emb_lookup(token.reshape(-1), table)
    return out.reshape(token.shape + (_D,))


# speedup vs baseline: 1.0016x; 1.0016x over previous
import functools

import jax
import jax.numpy as jnp
from jax import lax
from jax.experimental import pallas as pl
from jax.experimental.pallas import tpu as pltpu
from jax.experimental.pallas import tpu_sc as plsc

_VOCAB = 1_000_000
_D = 64
_NC, _NS, _L = 2, 16, 16
_NW = _NC * _NS
_BT = 16384 * 50
_BPW = _BT // _NW
_C = 400
_NCHUNK = _BPW // _C
_NBUF = 4
_NGRP = _NCHUNK // _NBUF

_mesh = plsc.VectorSubcoreMesh(core_axis_name="c", subcore_axis_name="s")


@functools.partial(
    pl.kernel,
    mesh=_mesh,
    out_type=jax.ShapeDtypeStruct((_BT, _D), jnp.float32),
    compiler_params=pltpu.CompilerParams(use_tc_tiling_on_sc=False),
    scratch_types=[
        pltpu.VMEM((_BPW,), jnp.int32),
        pltpu.VMEM((_C, _D), jnp.float32),
        pltpu.VMEM((_C, _D), jnp.float32),
        pltpu.VMEM((_C, _D), jnp.float32),
        pltpu.VMEM((_C, _D), jnp.float32),
        pltpu.SemaphoreType.DMA,
        pltpu.SemaphoreType.DMA,
        pltpu.SemaphoreType.DMA,
        pltpu.SemaphoreType.DMA,
        pltpu.SemaphoreType.DMA,
        pltpu.SemaphoreType.DMA,
        pltpu.SemaphoreType.DMA,
        pltpu.SemaphoreType.DMA,
    ],
)
def _emb_lookup(token_hbm, table_hbm, out_hbm, idx_v,
                b0, b1, b2, b3, g0, g1, g2, g3, s0, s1, s2, s3):
    bufs = (b0, b1, b2, b3)
    gsem = (g0, g1, g2, g3)
    ssem = (s0, s1, s2, s3)
    wid = lax.axis_index("s") * _NC + lax.axis_index("c")
    base = wid * _BPW
    _HEAD = (_NBUF - 1) * _C
    pltpu.async_copy(token_hbm.at[pl.ds(base, _HEAD)],
                     idx_v.at[pl.ds(0, _HEAD)], g3)
    pltpu.async_copy(token_hbm.at[pl.ds(base + _HEAD, _BPW - _HEAD)],
                     idx_v.at[pl.ds(_HEAD, _BPW - _HEAD)], s3)
    pltpu.make_async_copy(token_hbm.at[pl.ds(base, _HEAD)],
                          idx_v.at[pl.ds(0, _HEAD)], g3).wait()

    def clamp_span(lo, hi):
        def cb(j, _):
            s = j * _L
            v = idx_v[pl.ds(s, _L)]
            idx_v[pl.ds(s, _L)] = jnp.minimum(jnp.maximum(v, 0), _VOCAB - 1)
            return 0
        lax.fori_loop(lo // _L, hi // _L, cb, 0)

    def fire_gather(c, buf, sem):
        pltpu.async_copy(table_hbm.at[idx_v.at[pl.ds(c * _C, _C)]], buf, sem)

    def wait_gather(buf, sem):
        pltpu.make_async_copy(table_hbm.at[idx_v.at[pl.ds(0, _C)]], buf, sem).wait()

    def fire_store(c, buf, sem):
        pltpu.async_copy(buf, out_hbm.at[pl.ds(base + c * _C, _C)], sem)

    def wait_store(buf, sem):
        pltpu.make_async_copy(buf, out_hbm.at[pl.ds(base, _C)], sem).wait()

    clamp_span(0, _HEAD)
    for b in range(_NBUF - 1):
        fire_gather(b, bufs[b], gsem[b])
    pltpu.make_async_copy(token_hbm.at[pl.ds(base + _HEAD, _BPW - _HEAD)],
                          idx_v.at[pl.ds(_HEAD, _BPW - _HEAD)], s3).wait()
    clamp_span(_HEAD, _BPW)

    def body(g, _):
        i0 = g * _NBUF
        for b in range(_NBUF):
            i = i0 + b
            b2 = (b + _NBUF - 1) % _NBUF
            ahead = i + _NBUF - 1

            @pl.when(jnp.logical_and(ahead < _NCHUNK, i >= 1))
            def _drain_prev_store():
                wait_store(bufs[b2], ssem[b2])

            @pl.when(ahead < _NCHUNK)
            def _launch_ahead():
                fire_gather(ahead, bufs[b2], gsem[b2])

            wait_gather(bufs[b], gsem[b])
            fire_store(i, bufs[b], ssem[b])
        return 0

    lax.fori_loop(0, _NGRP, body, 0)
    for b in range(_NBUF):
        wait_store(bufs[b], ssem[b])


def kernel(token, table):
    out = _emb_lookup(token.reshape(-1), table)
    return out.reshape(token.shape + (_D,))
